# baseline (speedup 1.0000x reference)
import functools

import jax
import jax.numpy as jnp
from jax import lax
from jax.experimental import pallas as pl
from jax.experimental.pallas import tpu as pltpu
from jax.experimental.pallas import tpu_sc as plsc

B = 128
L = 1024
D = 96
KEEP = 512
P_IN = L + 1
P_OUT = KEEP + 1
NC, NS = 2, 16
NW = NC * NS
F_PER_W = D // NW
B_PER_S = B // NS
RING = 640
CHUNK = 64
NCHUNK = 17
OBLK = 64
KROWS = 576

_MININT_PY = -2**31


def _monotone_key(s):
    return s ^ (lax.shift_right_arithmetic(s, 31) & jnp.int32(0x7FFFFFFF))


def _select_body(noise_ref, thr_ref, need_ref):
    s = lax.bitcast_convert_type(noise_ref[...], jnp.int32)
    m = _monotone_key(s)
    u = m ^ jnp.int32(_MININT_PY)
    p = jnp.zeros((B, 1), jnp.int32)
    kk = jnp.full((B, 1), KEEP - 1, jnp.int32)
    for bit in range(31, -1, -1):
        ub = lax.shift_right_logical(u, bit)
        pb = lax.shift_right_logical(p, bit)
        cnt = jnp.sum((ub == pb).astype(jnp.int32), axis=1, keepdims=True)
        take = kk >= cnt
        bitval = jnp.int32(_MININT_PY if bit == 31 else 1 << bit)
        p = jnp.where(take, p | bitval, p)
        kk = jnp.where(take, kk - cnt, kk)
    t_m = p ^ jnp.int32(_MININT_PY)
    cnt_less = jnp.sum((m < t_m).astype(jnp.int32), axis=1, keepdims=True)
    need = KEEP - cnt_less
    thr_ref[...] = jnp.broadcast_to(t_m, (B, NS))
    need_ref[...] = jnp.broadcast_to(need, (B, NS))


def _tc_select(noise):
    return pl.pallas_call(
        _select_body,
        out_shape=[
            jax.ShapeDtypeStruct((B, NS), jnp.int32),
            jax.ShapeDtypeStruct((B, NS), jnp.int32),
        ],
    )(noise)


def _sc_body(noise_hbm, thr_hbm, need_hbm, xt_hbm, out_hbm,
             noise_v, thr_v, need_v, colblk, ring, kwin, obuf, ksp, sem):
    cid = lax.axis_index("c")
    sid = lax.axis_index("s")
    wid = sid * NC + cid
    lanes = lax.iota(jnp.int32, NS)

    pltpu.sync_copy(thr_hbm.at[pl.ds(sid * B_PER_S, B_PER_S)], thr_v)
    pltpu.sync_copy(need_hbm.at[pl.ds(sid * B_PER_S, B_PER_S)], need_v)
    for bb in range(B_PER_S):
        b = sid * B_PER_S + bb
        pltpu.sync_copy(noise_hbm.at[b], noise_v)
        t_vec = thr_v[bb]
        need_vec = need_v[bb]
        plsc.store_scatter(colblk, [lanes, jnp.full((NS,), bb, jnp.int32)],
                           jnp.zeros((NS,), jnp.int32), mask=lanes < 1)

        def chunk(k, carry):
            seq, ties = carry
            v = noise_v[k >> 3, pl.ds((k & 7) * NS, NS)]
            m = _monotone_key(plsc.bitcast(v, jnp.int32))
            less = m < t_vec
            eq = m == t_vec
            eq_i = jnp.where(eq, 1, 0).astype(jnp.int32)
            cum_eq = plsc.cumsum(eq_i)
            tie_rank = (ties + cum_eq) - eq_i
            keep = jnp.logical_or(less, jnp.logical_and(eq, tie_rank < need_vec))
            keep_i = jnp.where(keep, 1, 0).astype(jnp.int32)
            slot = seq + plsc.cumsum(keep_i)
            plane = (1 + k * NS) + lanes
            plsc.store_scatter(colblk, [slot, jnp.full((NS,), bb, jnp.int32)],
                               plane, mask=keep)
            return seq + jnp.sum(keep_i), ties + jnp.sum(eq_i)

        lax.fori_loop(0, L // NS, chunk, (jnp.int32(0), jnp.int32(0)))

    pltpu.sync_copy(colblk, ksp.at[:, pl.ds(sid * B_PER_S, B_PER_S)])
    plsc.subcore_barrier()

    for ff in range(F_PER_W):
        f = wid * F_PER_W + ff

        def load_chunk(c, n):
            pltpu.sync_copy(
                xt_hbm.at[pl.ds(c * CHUNK, n), pl.ds(f, 1), :],
                ring.at[pl.ds((c % (RING // CHUNK)) * CHUNK, n)])

        for c in range(RING // CHUNK):
            load_chunk(c, CHUNK)
        for J in range(9):
            nrows = OBLK if J < 8 else P_OUT - 8 * OBLK
            pltpu.sync_copy(ksp.at[pl.ds(J * OBLK, OBLK)], kwin)

            def row(jj, _):
                def grp(g, _):
                    kv = kwin[jj, pl.ds(g * NS, NS)]
                    slot = jnp.where(kv >= RING, kv - RING, kv)
                    vals = plsc.load_gather(
                        ring, [slot, jnp.zeros((NS,), jnp.int32),
                               g * NS + lanes])
                    obuf[jj, 0, pl.ds(g * NS, NS)] = vals
                    return 0

                return lax.fori_loop(0, B // NS, grp, 0)

            lax.fori_loop(0, nrows, row, 0)
            pltpu.sync_copy(obuf.at[pl.ds(0, nrows)],
                            out_hbm.at[pl.ds(J * OBLK, nrows), pl.ds(f, 1), :])
            c = J + 10
            if c < NCHUNK:
                load_chunk(c, CHUNK if c < NCHUNK - 1
                           else P_IN - (NCHUNK - 1) * CHUNK)


def _sc_gather(noise, thr, need, x_t):
    mesh = plsc.VectorSubcoreMesh(core_axis_name="c", subcore_axis_name="s")
    k = functools.partial(
        pl.kernel,
        mesh=mesh,
        out_type=jax.ShapeDtypeStruct((P_OUT, D, B), jnp.float32),
        scratch_types=[
            pltpu.VMEM((L // 128, 128), jnp.float32),
            pltpu.VMEM((B_PER_S, NS), jnp.int32),
            pltpu.VMEM((B_PER_S, NS), jnp.int32),
            pltpu.VMEM((KROWS, B_PER_S), jnp.int32),
            pltpu.VMEM((RING, 1, B), jnp.float32),
            pltpu.VMEM((OBLK, B), jnp.int32),
            pltpu.VMEM((OBLK, 1, B), jnp.float32),
            pltpu.VMEM_SHARED((KROWS, B), jnp.int32),
            pltpu.SemaphoreType.DMA,
        ],
        compiler_params=pltpu.CompilerParams(
            needs_layout_passes=False, use_tc_tiling_on_sc=False),
    )(_sc_body)
    return k(noise, thr, need, x_t)


def kernel(x):
    noise = jax.random.normal(jax.random.key(1), (B, L), dtype=jnp.float32)
    thr, need = _tc_select(noise)
    x_t = jnp.transpose(x, (1, 2, 0))
    out_t = _sc_gather(noise.reshape(B, L // 128, 128), thr, need, x_t)
    return jnp.transpose(out_t, (2, 0, 1))


# speedup vs baseline: 1.5372x; 1.5372x over previous
import functools

import jax
import jax.numpy as jnp
from jax import lax
from jax.experimental import pallas as pl
from jax.experimental.pallas import tpu as pltpu
from jax.experimental.pallas import tpu_sc as plsc

B = 128
L = 1024
D = 96
KEEP = 512
P_IN = L + 1
P_OUT = KEEP + 1
NC, NS = 2, 16
NW = NC * NS
F_PER_W = D // NW
B_PER_S = B // NS
RING = 640
CHUNK = 64
NCHUNK = 17
OBLK = 64
KROWS = 576

_MININT_PY = -2**31


def _monotone_key(s):
    return s ^ (lax.shift_right_arithmetic(s, 31) & jnp.int32(0x7FFFFFFF))


def _select_body(noise_ref, thr_ref, need_ref):
    s = lax.bitcast_convert_type(noise_ref[...], jnp.int32)
    m = _monotone_key(s)
    u = m ^ jnp.int32(_MININT_PY)
    p = jnp.zeros((B, 1), jnp.int32)
    kk = jnp.full((B, 1), KEEP - 1, jnp.int32)
    for bit in range(31, -1, -1):
        ub = lax.shift_right_logical(u, bit)
        pb = lax.shift_right_logical(p, bit)
        cnt = jnp.sum((ub == pb).astype(jnp.int32), axis=1, keepdims=True)
        take = kk >= cnt
        bitval = jnp.int32(_MININT_PY if bit == 31 else 1 << bit)
        p = jnp.where(take, p | bitval, p)
        kk = jnp.where(take, kk - cnt, kk)
    t_m = p ^ jnp.int32(_MININT_PY)
    cnt_less = jnp.sum((m < t_m).astype(jnp.int32), axis=1, keepdims=True)
    need = KEEP - cnt_less
    thr_ref[...] = jnp.broadcast_to(t_m, (B, NS))
    need_ref[...] = jnp.broadcast_to(need, (B, NS))


def _tc_select(noise):
    return pl.pallas_call(
        _select_body,
        out_shape=[
            jax.ShapeDtypeStruct((B, NS), jnp.int32),
            jax.ShapeDtypeStruct((B, NS), jnp.int32),
        ],
    )(noise)


def _sc_body(noise_hbm, thr_hbm, need_hbm, xt_hbm, out_hbm,
             noise_v, thr_v, need_v, colblk, ring, kwin, obuf, ksp, sem):
    cid = lax.axis_index("c")
    sid = lax.axis_index("s")
    wid = sid * NC + cid
    lanes = lax.iota(jnp.int32, NS)

    pltpu.sync_copy(thr_hbm.at[pl.ds(sid * B_PER_S, B_PER_S)], thr_v)
    pltpu.sync_copy(need_hbm.at[pl.ds(sid * B_PER_S, B_PER_S)], need_v)
    for bb in range(B_PER_S):
        b = sid * B_PER_S + bb
        pltpu.sync_copy(noise_hbm.at[b], noise_v)
        t_vec = thr_v[bb]
        need_vec = need_v[bb]
        plsc.store_scatter(colblk, [lanes, jnp.full((NS,), bb, jnp.int32)],
                           jnp.zeros((NS,), jnp.int32), mask=lanes < 1)

        def chunk(k, carry):
            seq, ties = carry
            v = noise_v[k >> 3, pl.ds((k & 7) * NS, NS)]
            m = _monotone_key(plsc.bitcast(v, jnp.int32))
            less = m < t_vec
            eq = m == t_vec
            eq_i = jnp.where(eq, 1, 0).astype(jnp.int32)
            cum_eq = plsc.cumsum(eq_i)
            tie_rank = (ties + cum_eq) - eq_i
            keep = jnp.logical_or(less, jnp.logical_and(eq, tie_rank < need_vec))
            keep_i = jnp.where(keep, 1, 0).astype(jnp.int32)
            slot = seq + plsc.cumsum(keep_i)
            plane = (1 + k * NS) + lanes
            plsc.store_scatter(colblk, [slot, jnp.full((NS,), bb, jnp.int32)],
                               plane, mask=keep)
            return seq + jnp.sum(keep_i), ties + jnp.sum(eq_i)

        lax.fori_loop(0, L // NS, chunk, (jnp.int32(0), jnp.int32(0)))

    pltpu.sync_copy(colblk, ksp.at[:, pl.ds(sid * B_PER_S, B_PER_S)])
    plsc.subcore_barrier()

    zeros16 = jnp.zeros((NS,), jnp.int32)
    lane_g = [g * NS + lanes for g in range(B // NS)]
    for ff in range(F_PER_W):
        f = wid * F_PER_W + ff

        def load_chunk(c, n):
            pltpu.sync_copy(
                xt_hbm.at[pl.ds(c * CHUNK, n), pl.ds(f, 1), :],
                ring.at[pl.ds((c % (RING // CHUNK)) * CHUNK, n)])

        for c in range(RING // CHUNK):
            load_chunk(c, CHUNK)
        for J in range(9):
            nrows = OBLK if J < 8 else P_OUT - 8 * OBLK
            pltpu.sync_copy(ksp.at[pl.ds(J * OBLK, OBLK)], kwin)

            def row(jj, _):
                for g in range(B // NS):
                    kv = kwin[jj, pl.ds(g * NS, NS)]
                    slot = jnp.where(kv >= RING, kv - RING, kv)
                    vals = plsc.load_gather(ring, [slot, zeros16,
                                                   lane_g[g]])
                    obuf[jj, 0, pl.ds(g * NS, NS)] = vals
                return 0

            lax.fori_loop(0, nrows, row, 0)
            pltpu.sync_copy(obuf.at[pl.ds(0, nrows)],
                            out_hbm.at[pl.ds(J * OBLK, nrows), pl.ds(f, 1), :])
            c = J + 10
            if c < NCHUNK:
                load_chunk(c, CHUNK if c < NCHUNK - 1
                           else P_IN - (NCHUNK - 1) * CHUNK)


def _sc_gather(noise, thr, need, x_t):
    mesh = plsc.VectorSubcoreMesh(core_axis_name="c", subcore_axis_name="s")
    k = functools.partial(
        pl.kernel,
        mesh=mesh,
        out_type=jax.ShapeDtypeStruct((P_OUT, D, B), jnp.float32),
        scratch_types=[
            pltpu.VMEM((L // 128, 128), jnp.float32),
            pltpu.VMEM((B_PER_S, NS), jnp.int32),
            pltpu.VMEM((B_PER_S, NS), jnp.int32),
            pltpu.VMEM((KROWS, B_PER_S), jnp.int32),
            pltpu.VMEM((RING, 1, B), jnp.float32),
            pltpu.VMEM((OBLK, B), jnp.int32),
            pltpu.VMEM((OBLK, 1, B), jnp.float32),
            pltpu.VMEM_SHARED((KROWS, B), jnp.int32),
            pltpu.SemaphoreType.DMA,
        ],
        compiler_params=pltpu.CompilerParams(
            needs_layout_passes=False, use_tc_tiling_on_sc=False),
    )(_sc_body)
    return k(noise, thr, need, x_t)


def kernel(x):
    noise = jax.random.normal(jax.random.key(1), (B, L), dtype=jnp.float32)
    thr, need = _tc_select(noise)
    x_t = jnp.transpose(x, (1, 2, 0))
    out_t = _sc_gather(noise.reshape(B, L // 128, 128), thr, need, x_t)
    return jnp.transpose(out_t, (2, 0, 1))


# speedup vs baseline: 2.2035x; 1.4334x over previous
import functools

import jax
import jax.numpy as jnp
from jax import lax
from jax.experimental import pallas as pl
from jax.experimental.pallas import tpu as pltpu
from jax.experimental.pallas import tpu_sc as plsc

B = 128
L = 1024
D = 96
KEEP = 512
P_IN = L + 1
P_OUT = KEEP + 1
NC, NS = 2, 16
NW = NC * NS
F_PER_W = D // NW
B_PER_S = B // NS
RING = 608
CHUNK = 32
NCHUNK = 33
OBLK = 32
NBLK = 17
KROWS = 576

_MININT_PY = -2**31


def _monotone_key(s):
    return s ^ (lax.shift_right_arithmetic(s, 31) & jnp.int32(0x7FFFFFFF))


def _select_body(noise_ref, thr_ref, need_ref):
    s = lax.bitcast_convert_type(noise_ref[...], jnp.int32)
    m = _monotone_key(s)
    u = m ^ jnp.int32(_MININT_PY)
    p = jnp.zeros((B, 1), jnp.int32)
    kk = jnp.full((B, 1), KEEP - 1, jnp.int32)
    for bit in range(31, -1, -1):
        ub = lax.shift_right_logical(u, bit)
        pb = lax.shift_right_logical(p, bit)
        cnt = jnp.sum((ub == pb).astype(jnp.int32), axis=1, keepdims=True)
        take = kk >= cnt
        bitval = jnp.int32(_MININT_PY if bit == 31 else 1 << bit)
        p = jnp.where(take, p | bitval, p)
        kk = jnp.where(take, kk - cnt, kk)
    t_m = p ^ jnp.int32(_MININT_PY)
    cnt_less = jnp.sum((m < t_m).astype(jnp.int32), axis=1, keepdims=True)
    need = KEEP - cnt_less
    thr_ref[...] = jnp.broadcast_to(t_m, (B, NS))
    need_ref[...] = jnp.broadcast_to(need, (B, NS))


def _tc_select(noise):
    return pl.pallas_call(
        _select_body,
        out_shape=[
            jax.ShapeDtypeStruct((B, NS), jnp.int32),
            jax.ShapeDtypeStruct((B, NS), jnp.int32),
        ],
    )(noise)


def _sc_body(noise_hbm, thr_hbm, need_hbm, xt_hbm, out_hbm,
             noise_v, thr_v, need_v, colblk, ring, kwin, obuf, ksp,
             sem_in, sem_k, sem_out):
    cid = lax.axis_index("c")
    sid = lax.axis_index("s")
    wid = sid * NC + cid
    lanes = lax.iota(jnp.int32, NS)

    pltpu.sync_copy(thr_hbm.at[pl.ds(sid * B_PER_S, B_PER_S)], thr_v)
    pltpu.sync_copy(need_hbm.at[pl.ds(sid * B_PER_S, B_PER_S)], need_v)
    for bb in range(B_PER_S):
        b = sid * B_PER_S + bb
        pltpu.sync_copy(noise_hbm.at[b], noise_v)
        t_vec = thr_v[bb]
        need_vec = need_v[bb]
        plsc.store_scatter(colblk, [lanes, jnp.full((NS,), bb, jnp.int32)],
                           jnp.zeros((NS,), jnp.int32), mask=lanes < 1)

        def chunk(k, carry):
            seq, ties = carry
            v = noise_v[k >> 3, pl.ds((k & 7) * NS, NS)]
            m = _monotone_key(plsc.bitcast(v, jnp.int32))
            less = m < t_vec
            eq = m == t_vec
            eq_i = jnp.where(eq, 1, 0).astype(jnp.int32)
            cum_eq = plsc.cumsum(eq_i)
            tie_rank = (ties + cum_eq) - eq_i
            keep = jnp.logical_or(less, jnp.logical_and(eq, tie_rank < need_vec))
            keep_i = jnp.where(keep, 1, 0).astype(jnp.int32)
            slot = seq + plsc.cumsum(keep_i)
            plane = (1 + k * NS) + lanes
            plsc.store_scatter(colblk, [slot, jnp.full((NS,), bb, jnp.int32)],
                               plane, mask=keep)
            return seq + jnp.sum(keep_i), ties + jnp.sum(eq_i)

        lax.fori_loop(0, L // NS, chunk, (jnp.int32(0), jnp.int32(0)))

    pltpu.sync_copy(colblk, ksp.at[:, pl.ds(sid * B_PER_S, B_PER_S)])
    plsc.subcore_barrier()

    zeros16 = jnp.zeros((NS,), jnp.int32)
    lane_g = [g * NS + lanes for g in range(B // NS)]
    for ff in range(F_PER_W):
        f = wid * F_PER_W + ff
        hin = {}

        def fire_chunk(c):
            n = CHUNK if c < NCHUNK - 1 else P_IN - (NCHUNK - 1) * CHUNK
            hin[c] = pltpu.async_copy(
                xt_hbm.at[pl.ds(c * CHUNK, n), pl.ds(f, 1), :],
                ring.at[pl.ds((c % (RING // CHUNK)) * CHUNK, n)], sem_in)

        hk = [None] * NBLK

        def fire_kwin(J):
            hk[J] = pltpu.async_copy(ksp.at[pl.ds(J * OBLK, OBLK)],
                                     kwin.at[J % 2], sem_k)

        for c in range(17):
            fire_chunk(c)
        fire_kwin(0)
        for c in range(17):
            hin[c].wait()
        hout = [None] * NBLK
        for J in range(NBLK):
            nrows = OBLK if J < NBLK - 1 else P_OUT - (NBLK - 1) * OBLK
            hk[J].wait()
            if J + 1 < NBLK:
                fire_kwin(J + 1)
            if J >= 1:
                hin[J + 16].wait()
            if J + 17 < NCHUNK:
                fire_chunk(J + 17)
            if J >= 2:
                hout[J - 2].wait()
            kb = J % 2

            def row(jj, _):
                for g in range(B // NS):
                    kv = kwin[kb, jj, pl.ds(g * NS, NS)]
                    slot = jnp.where(kv >= RING, kv - RING, kv)
                    vals = plsc.load_gather(ring, [slot, zeros16, lane_g[g]])
                    obuf[kb, jj, 0, pl.ds(g * NS, NS)] = vals
                return 0

            lax.fori_loop(0, nrows, row, 0)
            hout[J] = pltpu.async_copy(
                obuf.at[kb].at[pl.ds(0, nrows)],
                out_hbm.at[pl.ds(J * OBLK, nrows), pl.ds(f, 1), :], sem_out)
        hout[NBLK - 2].wait()
        hout[NBLK - 1].wait()


def _sc_gather(noise, thr, need, x_t):
    mesh = plsc.VectorSubcoreMesh(core_axis_name="c", subcore_axis_name="s")
    k = functools.partial(
        pl.kernel,
        mesh=mesh,
        out_type=jax.ShapeDtypeStruct((P_OUT, D, B), jnp.float32),
        scratch_types=[
            pltpu.VMEM((L // 128, 128), jnp.float32),
            pltpu.VMEM((B_PER_S, NS), jnp.int32),
            pltpu.VMEM((B_PER_S, NS), jnp.int32),
            pltpu.VMEM((KROWS, B_PER_S), jnp.int32),
            pltpu.VMEM((RING, 1, B), jnp.float32),
            pltpu.VMEM((2, OBLK, B), jnp.int32),
            pltpu.VMEM((2, OBLK, 1, B), jnp.float32),
            pltpu.VMEM_SHARED((KROWS, B), jnp.int32),
            pltpu.SemaphoreType.DMA,
            pltpu.SemaphoreType.DMA,
            pltpu.SemaphoreType.DMA,
        ],
        compiler_params=pltpu.CompilerParams(
            needs_layout_passes=False, use_tc_tiling_on_sc=False),
    )(_sc_body)
    return k(noise, thr, need, x_t)


def kernel(x):
    noise = jax.random.normal(jax.random.key(1), (B, L), dtype=jnp.float32)
    thr, need = _tc_select(noise)
    x_t = jnp.transpose(x, (1, 2, 0))
    out_t = _sc_gather(noise.reshape(B, L // 128, 128), thr, need, x_t)
    return jnp.transpose(out_t, (2, 0, 1))


# speedup vs baseline: 2.2667x; 1.0287x over previous
import functools

import jax
import jax.numpy as jnp
from jax import lax
from jax.experimental import pallas as pl
from jax.experimental.pallas import tpu as pltpu
from jax.experimental.pallas import tpu_sc as plsc

B = 128
L = 1024
D = 96
KEEP = 512
P_IN = L + 1
P_OUT = KEEP + 1
NC, NS = 2, 16
NW = NC * NS
F_PER_W = D // NW
B_PER_S = B // NS
RING = 640
CHUNK = 64
NCHUNK = 17
OBLK = 64
NBLK = 9
KROWS = 576

_MININT_PY = -2**31


def _monotone_key(s):
    return s ^ (lax.shift_right_arithmetic(s, 31) & jnp.int32(0x7FFFFFFF))


def _select_body(noise_ref, thr_ref, need_ref):
    s = lax.bitcast_convert_type(noise_ref[...], jnp.int32)
    m = _monotone_key(s)
    u = m ^ jnp.int32(_MININT_PY)
    p = jnp.zeros((B, 1), jnp.int32)
    kk = jnp.full((B, 1), KEEP - 1, jnp.int32)
    for bit in range(31, -1, -1):
        ub = lax.shift_right_logical(u, bit)
        pb = lax.shift_right_logical(p, bit)
        cnt = jnp.sum((ub == pb).astype(jnp.int32), axis=1, keepdims=True)
        take = kk >= cnt
        bitval = jnp.int32(_MININT_PY if bit == 31 else 1 << bit)
        p = jnp.where(take, p | bitval, p)
        kk = jnp.where(take, kk - cnt, kk)
    t_m = p ^ jnp.int32(_MININT_PY)
    cnt_less = jnp.sum((m < t_m).astype(jnp.int32), axis=1, keepdims=True)
    need = KEEP - cnt_less
    thr_ref[...] = jnp.broadcast_to(t_m, (B, NS))
    need_ref[...] = jnp.broadcast_to(need, (B, NS))


def _tc_select(noise):
    return pl.pallas_call(
        _select_body,
        out_shape=[
            jax.ShapeDtypeStruct((B, NS), jnp.int32),
            jax.ShapeDtypeStruct((B, NS), jnp.int32),
        ],
    )(noise)


def _sc_body(noise_hbm, thr_hbm, need_hbm, xt_hbm, out_hbm,
             noise_v, thr_v, need_v, colblk, ring, kwin, obuf, ksp,
             sem_in, sem_k, sem_out):
    cid = lax.axis_index("c")
    sid = lax.axis_index("s")
    wid = sid * NC + cid
    lanes = lax.iota(jnp.int32, NS)

    pltpu.sync_copy(thr_hbm.at[pl.ds(sid * B_PER_S, B_PER_S)], thr_v)
    pltpu.sync_copy(need_hbm.at[pl.ds(sid * B_PER_S, B_PER_S)], need_v)
    for bb in range(B_PER_S):
        b = sid * B_PER_S + bb
        pltpu.sync_copy(noise_hbm.at[b], noise_v)
        t_vec = thr_v[bb]
        need_vec = need_v[bb]
        plsc.store_scatter(colblk, [lanes, jnp.full((NS,), bb, jnp.int32)],
                           jnp.zeros((NS,), jnp.int32), mask=lanes < 1)

        def chunk(k, carry):
            seq, ties = carry
            v = noise_v[k >> 3, pl.ds((k & 7) * NS, NS)]
            m = _monotone_key(plsc.bitcast(v, jnp.int32))
            less = m < t_vec
            eq = m == t_vec
            eq_i = jnp.where(eq, 1, 0).astype(jnp.int32)
            cum_eq = plsc.cumsum(eq_i)
            tie_rank = (ties + cum_eq) - eq_i
            keep = jnp.logical_or(less, jnp.logical_and(eq, tie_rank < need_vec))
            keep_i = jnp.where(keep, 1, 0).astype(jnp.int32)
            slot = seq + plsc.cumsum(keep_i)
            plane = (1 + k * NS) + lanes
            plsc.store_scatter(colblk, [slot, jnp.full((NS,), bb, jnp.int32)],
                               plane, mask=keep)
            return seq + jnp.sum(keep_i), ties + jnp.sum(eq_i)

        lax.fori_loop(0, L // NS, chunk, (jnp.int32(0), jnp.int32(0)))

    pltpu.sync_copy(colblk, ksp.at[:, pl.ds(sid * B_PER_S, B_PER_S)])
    plsc.subcore_barrier()

    zeros16 = jnp.zeros((NS,), jnp.int32)
    lane_g = [g * NS + lanes for g in range(B // NS)]
    for ff in range(F_PER_W):
        f = wid * F_PER_W + ff
        hin = {}

        def fire_chunk(c):
            n = CHUNK if c < NCHUNK - 1 else P_IN - (NCHUNK - 1) * CHUNK
            hin[c] = pltpu.async_copy(
                xt_hbm.at[pl.ds(c * CHUNK, n), pl.ds(f, 1), :],
                ring.at[pl.ds((c % (RING // CHUNK)) * CHUNK, n)], sem_in)

        hk = [None] * NBLK

        def fire_kwin(J):
            hk[J] = pltpu.async_copy(ksp.at[pl.ds(J * OBLK, OBLK)],
                                     kwin.at[J % 2], sem_k)

        for c in range(9):
            fire_chunk(c)
        fire_kwin(0)
        for c in range(9):
            hin[c].wait()
        hout = [None] * NBLK
        for J in range(NBLK):
            nrows = OBLK if J < NBLK - 1 else P_OUT - (NBLK - 1) * OBLK
            hk[J].wait()
            if J + 1 < NBLK:
                fire_kwin(J + 1)
            if J >= 1:
                hin[J + 8].wait()
            if J + 9 < NCHUNK:
                fire_chunk(J + 9)
            if J >= 2:
                hout[J - 2].wait()
            kb = J % 2

            def row(jj, _):
                for g in range(B // NS):
                    kv = kwin[kb, jj, pl.ds(g * NS, NS)]
                    slot = jnp.where(kv >= RING, kv - RING, kv)
                    vals = plsc.load_gather(ring, [slot, zeros16, lane_g[g]])
                    obuf[kb, jj, 0, pl.ds(g * NS, NS)] = vals
                return 0

            lax.fori_loop(0, nrows, row, 0)
            hout[J] = pltpu.async_copy(
                obuf.at[kb].at[pl.ds(0, nrows)],
                out_hbm.at[pl.ds(J * OBLK, nrows), pl.ds(f, 1), :], sem_out)
        hout[NBLK - 2].wait()
        hout[NBLK - 1].wait()


def _sc_gather(noise, thr, need, x_t):
    mesh = plsc.VectorSubcoreMesh(core_axis_name="c", subcore_axis_name="s")
    k = functools.partial(
        pl.kernel,
        mesh=mesh,
        out_type=jax.ShapeDtypeStruct((P_OUT, D, B), jnp.float32),
        scratch_types=[
            pltpu.VMEM((L // 128, 128), jnp.float32),
            pltpu.VMEM((B_PER_S, NS), jnp.int32),
            pltpu.VMEM((B_PER_S, NS), jnp.int32),
            pltpu.VMEM((KROWS, B_PER_S), jnp.int32),
            pltpu.VMEM((RING, 1, B), jnp.float32),
            pltpu.VMEM((2, OBLK, B), jnp.int32),
            pltpu.VMEM((2, OBLK, 1, B), jnp.float32),
            pltpu.VMEM_SHARED((KROWS, B), jnp.int32),
            pltpu.SemaphoreType.DMA,
            pltpu.SemaphoreType.DMA,
            pltpu.SemaphoreType.DMA,
        ],
        compiler_params=pltpu.CompilerParams(
            needs_layout_passes=False, use_tc_tiling_on_sc=False),
    )(_sc_body)
    return k(noise, thr, need, x_t)


def kernel(x):
    noise = jax.random.normal(jax.random.key(1), (B, L), dtype=jnp.float32)
    thr, need = _tc_select(noise)
    x_t = jnp.transpose(x, (1, 2, 0))
    out_t = _sc_gather(noise.reshape(B, L // 128, 128), thr, need, x_t)
    return jnp.transpose(out_t, (2, 0, 1))


# speedup vs baseline: 2.3621x; 1.0421x over previous
import functools

import jax
import jax.numpy as jnp
from jax import lax
from jax.experimental import pallas as pl
from jax.experimental.pallas import tpu as pltpu
from jax.experimental.pallas import tpu_sc as plsc

B = 128
L = 1024
D = 96
KEEP = 512
P_IN = L + 1
P_OUT = KEEP + 1
NC, NS = 2, 16
NW = NC * NS
F_PER_W = D // NW
B_PER_S = B // NS
RING = 640
CHUNK = 64
NCHUNK = 17
OBLK = 64
NBLK = 9
KROWS = 576

_MININT_PY = -2**31


def _monotone_key(s):
    return s ^ (lax.shift_right_arithmetic(s, 31) & jnp.int32(0x7FFFFFFF))


def _select_body(noise_ref, thr_ref, need_ref):
    s = lax.bitcast_convert_type(noise_ref[...], jnp.int32)
    m = _monotone_key(s)
    u = m ^ jnp.int32(_MININT_PY)
    p = jnp.zeros((B, 1), jnp.int32)
    kk = jnp.full((B, 1), KEEP - 1, jnp.int32)
    for bit in range(31, -1, -1):
        ub = lax.shift_right_logical(u, bit)
        pb = lax.shift_right_logical(p, bit)
        cnt = jnp.sum((ub == pb).astype(jnp.int32), axis=1, keepdims=True)
        take = kk >= cnt
        bitval = jnp.int32(_MININT_PY if bit == 31 else 1 << bit)
        p = jnp.where(take, p | bitval, p)
        kk = jnp.where(take, kk - cnt, kk)
    t_m = p ^ jnp.int32(_MININT_PY)
    cnt_less = jnp.sum((m < t_m).astype(jnp.int32), axis=1, keepdims=True)
    need = KEEP - cnt_less
    eq = (m == t_m).astype(jnp.int32)
    c = eq
    for sh in (1, 2, 4, 8, 16, 32, 64, 128, 256, 512):
        c = c + jnp.concatenate(
            [jnp.zeros((B, sh), jnp.int32), c[:, :L - sh]], axis=1)
    pos = jax.lax.broadcasted_iota(jnp.int32, (B, L), 1)
    sel = jnp.logical_and(eq == 1, c == need)
    jstar = jnp.min(jnp.where(sel, pos, L), axis=1, keepdims=True)
    thr_ref[...] = jnp.broadcast_to(t_m, (B, NS))
    need_ref[...] = jnp.broadcast_to(jstar, (B, NS))


def _tc_select(noise):
    return pl.pallas_call(
        _select_body,
        out_shape=[
            jax.ShapeDtypeStruct((B, NS), jnp.int32),
            jax.ShapeDtypeStruct((B, NS), jnp.int32),
        ],
    )(noise)


def _sc_body(noise_hbm, thr_hbm, need_hbm, xt_hbm, out_hbm,
             noise_v, thr_v, need_v, colblk, ring, kwin, obuf, ksp,
             sem_in, sem_k, sem_out):
    cid = lax.axis_index("c")
    sid = lax.axis_index("s")
    wid = sid * NC + cid
    lanes = lax.iota(jnp.int32, NS)

    pltpu.sync_copy(thr_hbm.at[pl.ds(sid * B_PER_S, B_PER_S)], thr_v)
    pltpu.sync_copy(need_hbm.at[pl.ds(sid * B_PER_S, B_PER_S)], need_v)
    for bb in range(B_PER_S):
        b = sid * B_PER_S + bb
        pltpu.sync_copy(noise_hbm.at[b], noise_v)
        t_vec = thr_v[bb]
        jstar_vec = need_v[bb]
        plsc.store_scatter(colblk, [lanes, jnp.full((NS,), bb, jnp.int32)],
                           jnp.zeros((NS,), jnp.int32), mask=lanes < 1)

        bbvec = jnp.full((NS,), bb, jnp.int32)

        def chunk(k, seq):
            v = noise_v[k >> 3, pl.ds((k & 7) * NS, NS)]
            m = _monotone_key(plsc.bitcast(v, jnp.int32))
            tok = k * NS + lanes
            keep = jnp.logical_or(
                m < t_vec,
                jnp.logical_and(m == t_vec, tok <= jstar_vec))
            keep_i = jnp.where(keep, 1, 0).astype(jnp.int32)
            cum = plsc.cumsum(keep_i)
            slot = seq + cum
            plane = 1 + tok
            splane = jnp.where(plane >= RING, plane - RING, plane)
            plsc.store_scatter(colblk, [slot, bbvec], splane, mask=keep)
            return seq + cum[15]

        lax.fori_loop(0, L // NS, chunk, jnp.int32(0))

    pltpu.sync_copy(colblk, ksp.at[:, pl.ds(sid * B_PER_S, B_PER_S)])
    plsc.subcore_barrier()

    zeros16 = jnp.zeros((NS,), jnp.int32)
    lane_g = [g * NS + lanes for g in range(B // NS)]
    for ff in range(F_PER_W):
        f = wid * F_PER_W + ff
        hin = {}

        def fire_chunk(c):
            n = CHUNK if c < NCHUNK - 1 else P_IN - (NCHUNK - 1) * CHUNK
            hin[c] = pltpu.async_copy(
                xt_hbm.at[pl.ds(c * CHUNK, n), pl.ds(f, 1), :],
                ring.at[pl.ds((c % (RING // CHUNK)) * CHUNK, n)], sem_in)

        hk = [None] * NBLK

        def fire_kwin(J):
            hk[J] = pltpu.async_copy(ksp.at[pl.ds(J * OBLK, OBLK)],
                                     kwin.at[J % 2], sem_k)

        for c in range(9):
            fire_chunk(c)
        fire_kwin(0)
        for c in range(9):
            hin[c].wait()
        hout = [None] * NBLK
        for J in range(NBLK):
            nrows = OBLK if J < NBLK - 1 else P_OUT - (NBLK - 1) * OBLK
            hk[J].wait()
            if J + 1 < NBLK:
                fire_kwin(J + 1)
            if J >= 1:
                hin[J + 8].wait()
            if J + 9 < NCHUNK:
                fire_chunk(J + 9)
            if J >= 2:
                hout[J - 2].wait()
            kb = J % 2

            def row(jj, _):
                for g in range(B // NS):
                    slot = kwin[kb, jj, pl.ds(g * NS, NS)]
                    vals = plsc.load_gather(ring, [slot, zeros16, lane_g[g]])
                    obuf[kb, jj, 0, pl.ds(g * NS, NS)] = vals
                return 0

            lax.fori_loop(0, nrows, row, 0)
            hout[J] = pltpu.async_copy(
                obuf.at[kb].at[pl.ds(0, nrows)],
                out_hbm.at[pl.ds(J * OBLK, nrows), pl.ds(f, 1), :], sem_out)
        hout[NBLK - 2].wait()
        hout[NBLK - 1].wait()


def _sc_gather(noise, thr, need, x_t):
    mesh = plsc.VectorSubcoreMesh(core_axis_name="c", subcore_axis_name="s")
    k = functools.partial(
        pl.kernel,
        mesh=mesh,
        out_type=jax.ShapeDtypeStruct((P_OUT, D, B), jnp.float32),
        scratch_types=[
            pltpu.VMEM((L // 128, 128), jnp.float32),
            pltpu.VMEM((B_PER_S, NS), jnp.int32),
            pltpu.VMEM((B_PER_S, NS), jnp.int32),
            pltpu.VMEM((KROWS, B_PER_S), jnp.int32),
            pltpu.VMEM((RING, 1, B), jnp.float32),
            pltpu.VMEM((2, OBLK, B), jnp.int32),
            pltpu.VMEM((2, OBLK, 1, B), jnp.float32),
            pltpu.VMEM_SHARED((KROWS, B), jnp.int32),
            pltpu.SemaphoreType.DMA,
            pltpu.SemaphoreType.DMA,
            pltpu.SemaphoreType.DMA,
        ],
        compiler_params=pltpu.CompilerParams(
            needs_layout_passes=False, use_tc_tiling_on_sc=False),
    )(_sc_body)
    return k(noise, thr, need, x_t)


def kernel(x):
    noise = jax.random.normal(jax.random.key(1), (B, L), dtype=jnp.float32)
    thr, need = _tc_select(noise)
    x_t = jnp.transpose(x, (1, 2, 0))
    out_t = _sc_gather(noise.reshape(B, L // 128, 128), thr, need, x_t)
    return jnp.transpose(out_t, (2, 0, 1))


# speedup vs baseline: 2.3776x; 1.0066x over previous
import functools

import jax
import jax.numpy as jnp
from jax import lax
from jax.experimental import pallas as pl
from jax.experimental.pallas import tpu as pltpu
from jax.experimental.pallas import tpu_sc as plsc

B = 128
L = 1024
D = 96
KEEP = 512
P_IN = L + 1
P_OUT = KEEP + 1
NC, NS = 2, 16
NW = NC * NS
F_PER_W = D // NW
B_PER_S = B // NS
RING = 640
CHUNK = 64
NCHUNK = 17
OBLK = 64
NBLK = 9
KROWS = 576

_MININT_PY = -2**31


def _monotone_key(s):
    return s ^ (lax.shift_right_arithmetic(s, 31) & jnp.int32(0x7FFFFFFF))


def _select_body(noise_ref, thr_ref, need_ref):
    s = lax.bitcast_convert_type(noise_ref[...], jnp.int32)
    m = _monotone_key(s)
    u = m ^ jnp.int32(_MININT_PY)
    p = jnp.zeros((B, 1), jnp.int32)
    kk = jnp.full((B, 1), KEEP - 1, jnp.int32)
    for bit in range(31, -1, -1):
        ub = lax.shift_right_logical(u, bit)
        pb = lax.shift_right_logical(p, bit)
        cnt = jnp.sum((ub == pb).astype(jnp.int32), axis=1, keepdims=True)
        take = kk >= cnt
        bitval = jnp.int32(_MININT_PY if bit == 31 else 1 << bit)
        p = jnp.where(take, p | bitval, p)
        kk = jnp.where(take, kk - cnt, kk)
    t_m = p ^ jnp.int32(_MININT_PY)
    cnt_less = jnp.sum((m < t_m).astype(jnp.int32), axis=1, keepdims=True)
    need = KEEP - cnt_less
    eq = (m == t_m).astype(jnp.int32)
    c = eq
    for sh in (1, 2, 4, 8, 16, 32, 64, 128, 256, 512):
        c = c + jnp.concatenate(
            [jnp.zeros((B, sh), jnp.int32), c[:, :L - sh]], axis=1)
    pos = jax.lax.broadcasted_iota(jnp.int32, (B, L), 1)
    sel = jnp.logical_and(eq == 1, c == need)
    jstar = jnp.min(jnp.where(sel, pos, L), axis=1, keepdims=True)
    thr_ref[...] = jnp.broadcast_to(t_m, (B, NS))
    need_ref[...] = jnp.broadcast_to(jstar, (B, NS))


def _tc_select(noise):
    return pl.pallas_call(
        _select_body,
        out_shape=[
            jax.ShapeDtypeStruct((B, NS), jnp.int32),
            jax.ShapeDtypeStruct((B, NS), jnp.int32),
        ],
    )(noise)


def _sc_body(noise_hbm, thr_hbm, need_hbm, xt_hbm, out_hbm,
             noise_v, thr_v, need_v, colblk, ring, kwin, obuf, ksp,
             sem_in, sem_k, sem_out):
    cid = lax.axis_index("c")
    sid = lax.axis_index("s")
    wid = sid * NC + cid
    lanes = lax.iota(jnp.int32, NS)

    def fire_chunk_for(f, hin, c):
        n = CHUNK if c < NCHUNK - 1 else P_IN - (NCHUNK - 1) * CHUNK
        hin[c] = pltpu.async_copy(
            xt_hbm.at[pl.ds(c * CHUNK, n), pl.ds(f, 1), :],
            ring.at[pl.ds((c % (RING // CHUNK)) * CHUNK, n)], sem_in)

    hin0 = {}
    for c in range(9):
        fire_chunk_for(wid * F_PER_W, hin0, c)

    pltpu.sync_copy(thr_hbm.at[pl.ds(sid * B_PER_S, B_PER_S)], thr_v)
    pltpu.sync_copy(need_hbm.at[pl.ds(sid * B_PER_S, B_PER_S)], need_v)
    for bb in range(B_PER_S):
        b = sid * B_PER_S + bb
        pltpu.sync_copy(noise_hbm.at[b], noise_v)
        t_vec = thr_v[bb]
        jstar_vec = need_v[bb]
        plsc.store_scatter(colblk, [lanes, jnp.full((NS,), bb, jnp.int32)],
                           jnp.zeros((NS,), jnp.int32), mask=lanes < 1)

        bbvec = jnp.full((NS,), bb, jnp.int32)

        def chunk(k, seq):
            v = noise_v[k >> 3, pl.ds((k & 7) * NS, NS)]
            m = _monotone_key(plsc.bitcast(v, jnp.int32))
            tok = k * NS + lanes
            keep = jnp.logical_or(
                m < t_vec,
                jnp.logical_and(m == t_vec, tok <= jstar_vec))
            keep_i = jnp.where(keep, 1, 0).astype(jnp.int32)
            cum = plsc.cumsum(keep_i)
            slot = seq + cum
            plane = 1 + tok
            splane = jnp.where(plane >= RING, plane - RING, plane)
            plsc.store_scatter(colblk, [slot, bbvec], splane, mask=keep)
            return seq + cum[15]

        lax.fori_loop(0, L // NS, chunk, jnp.int32(0))

    pltpu.sync_copy(colblk, ksp.at[:, pl.ds(sid * B_PER_S, B_PER_S)])
    plsc.subcore_barrier()

    zeros16 = jnp.zeros((NS,), jnp.int32)
    lane_g = [g * NS + lanes for g in range(B // NS)]
    for ff in range(F_PER_W):
        f = wid * F_PER_W + ff
        hin = hin0 if ff == 0 else {}

        def fire_chunk(c, hin=hin, f=f):
            fire_chunk_for(f, hin, c)

        hk = [None] * NBLK

        def fire_kwin(J):
            hk[J] = pltpu.async_copy(ksp.at[pl.ds(J * OBLK, OBLK)],
                                     kwin.at[J % 2], sem_k)

        if ff != 0:
            for c in range(9):
                fire_chunk(c)
        fire_kwin(0)
        for c in range(9):
            hin[c].wait()
        hout = [None] * NBLK
        for J in range(NBLK):
            nrows = OBLK if J < NBLK - 1 else P_OUT - (NBLK - 1) * OBLK
            hk[J].wait()
            if J + 1 < NBLK:
                fire_kwin(J + 1)
            if J >= 1:
                hin[J + 8].wait()
            if J + 9 < NCHUNK:
                fire_chunk(J + 9)
            if J >= 2:
                hout[J - 2].wait()
            kb = J % 2

            def one_row(jj):
                for g in range(B // NS):
                    slot = kwin[kb, jj, pl.ds(g * NS, NS)]
                    vals = plsc.load_gather(ring, [slot, zeros16, lane_g[g]])
                    obuf[kb, jj, 0, pl.ds(g * NS, NS)] = vals

            if nrows == OBLK:
                def row2(jj, _):
                    one_row(2 * jj)
                    one_row(2 * jj + 1)
                    return 0

                lax.fori_loop(0, OBLK // 2, row2, 0)
            else:
                def row1(jj, _):
                    one_row(jj)
                    return 0

                lax.fori_loop(0, nrows, row1, 0)
            hout[J] = pltpu.async_copy(
                obuf.at[kb].at[pl.ds(0, nrows)],
                out_hbm.at[pl.ds(J * OBLK, nrows), pl.ds(f, 1), :], sem_out)
        hout[NBLK - 2].wait()
        hout[NBLK - 1].wait()


def _sc_gather(noise, thr, need, x_t):
    mesh = plsc.VectorSubcoreMesh(core_axis_name="c", subcore_axis_name="s")
    k = functools.partial(
        pl.kernel,
        mesh=mesh,
        out_type=jax.ShapeDtypeStruct((P_OUT, D, B), jnp.float32),
        scratch_types=[
            pltpu.VMEM((L // 128, 128), jnp.float32),
            pltpu.VMEM((B_PER_S, NS), jnp.int32),
            pltpu.VMEM((B_PER_S, NS), jnp.int32),
            pltpu.VMEM((KROWS, B_PER_S), jnp.int32),
            pltpu.VMEM((RING, 1, B), jnp.float32),
            pltpu.VMEM((2, OBLK, B), jnp.int32),
            pltpu.VMEM((2, OBLK, 1, B), jnp.float32),
            pltpu.VMEM_SHARED((KROWS, B), jnp.int32),
            pltpu.SemaphoreType.DMA,
            pltpu.SemaphoreType.DMA,
            pltpu.SemaphoreType.DMA,
        ],
        compiler_params=pltpu.CompilerParams(
            needs_layout_passes=False, use_tc_tiling_on_sc=False),
    )(_sc_body)
    return k(noise, thr, need, x_t)


def kernel(x):
    noise = jax.random.normal(jax.random.key(1), (B, L), dtype=jnp.float32)
    thr, need = _tc_select(noise)
    x_t = jnp.transpose(x, (1, 2, 0))
    out_t = _sc_gather(noise.reshape(B, L // 128, 128), thr, need, x_t)
    return jnp.transpose(out_t, (2, 0, 1))


# speedup vs baseline: 2.4131x; 1.0149x over previous
import functools

import jax
import jax.numpy as jnp
from jax import lax
from jax.experimental import pallas as pl
from jax.experimental.pallas import tpu as pltpu
from jax.experimental.pallas import tpu_sc as plsc

B = 128
L = 1024
D = 96
KEEP = 512
P_IN = L + 1
P_OUT = KEEP + 1
NC, NS = 2, 16
NW = NC * NS
F_PER_W = D // NW
B_PER_S = B // NS
RING = 640
CHUNK = 64
NCHUNK = 17
OBLK = 64
NBLK = 9
KROWS = 576

_MININT_PY = -2**31


def _monotone_key(s):
    return s ^ (lax.shift_right_arithmetic(s, 31) & jnp.int32(0x7FFFFFFF))


def _select_body(noise_ref, thr_ref, need_ref):
    s = lax.bitcast_convert_type(noise_ref[...], jnp.int32)
    m = _monotone_key(s)
    u = m ^ jnp.int32(_MININT_PY)
    p = jnp.zeros((B, 1), jnp.int32)
    kk = jnp.full((B, 1), KEEP - 1, jnp.int32)
    for bit in range(31, -1, -1):
        ub = lax.shift_right_logical(u, bit)
        pb = lax.shift_right_logical(p, bit)
        cnt = jnp.sum((ub == pb).astype(jnp.int32), axis=1, keepdims=True)
        take = kk >= cnt
        bitval = jnp.int32(_MININT_PY if bit == 31 else 1 << bit)
        p = jnp.where(take, p | bitval, p)
        kk = jnp.where(take, kk - cnt, kk)
    t_m = p ^ jnp.int32(_MININT_PY)
    cnt_less = jnp.sum((m < t_m).astype(jnp.int32), axis=1, keepdims=True)
    need = KEEP - cnt_less
    eq = (m == t_m).astype(jnp.int32)
    c = eq
    for sh in (1, 2, 4, 8, 16, 32, 64, 128, 256, 512):
        c = c + jnp.concatenate(
            [jnp.zeros((B, sh), jnp.int32), c[:, :L - sh]], axis=1)
    pos = jax.lax.broadcasted_iota(jnp.int32, (B, L), 1)
    sel = jnp.logical_and(eq == 1, c == need)
    jstar = jnp.min(jnp.where(sel, pos, L), axis=1, keepdims=True)
    thr_ref[...] = jnp.broadcast_to(t_m, (B, NS))
    need_ref[...] = jnp.broadcast_to(jstar, (B, NS))


def _tc_select(noise):
    return pl.pallas_call(
        _select_body,
        out_shape=[
            jax.ShapeDtypeStruct((B, NS), jnp.int32),
            jax.ShapeDtypeStruct((B, NS), jnp.int32),
        ],
    )(noise)


def _sc_body(noise_hbm, thr_hbm, need_hbm, xt_hbm, out_hbm,
             noise_v, thr_v, need_v, colblk, ring, kwin, obuf, ksp,
             sem_in, sem_k, sem_out):
    cid = lax.axis_index("c")
    sid = lax.axis_index("s")
    wid = sid * NC + cid
    lanes = lax.iota(jnp.int32, NS)

    def fire_chunk_for(f, hin, c):
        n = CHUNK if c < NCHUNK - 1 else P_IN - (NCHUNK - 1) * CHUNK
        hin[c] = pltpu.async_copy(
            xt_hbm.at[pl.ds(c * CHUNK, n), pl.ds(f, 1), :],
            ring.at[pl.ds((c % (RING // CHUNK)) * CHUNK, n)], sem_in)

    hin0 = {}
    for c in range(9):
        fire_chunk_for(wid * F_PER_W, hin0, c)

    pltpu.sync_copy(thr_hbm.at[pl.ds(sid * B_PER_S, B_PER_S)], thr_v)
    pltpu.sync_copy(need_hbm.at[pl.ds(sid * B_PER_S, B_PER_S)], need_v)
    plsc.store_scatter(colblk, [jnp.zeros((NS,), jnp.int32), lanes],
                       jnp.zeros((NS,), jnp.int32), mask=lanes < B_PER_S)
    for bb0 in range(0, B_PER_S, 2):
        pltpu.sync_copy(noise_hbm.at[pl.ds(sid * B_PER_S + bb0, 2)], noise_v)
        tv = [thr_v[bb0 + i] for i in range(2)]
        jv = [need_v[bb0 + i] for i in range(2)]
        bv = [jnp.full((NS,), bb0 + i, jnp.int32) for i in range(2)]

        def chunk(k, carry):
            tok = k * NS + lanes
            plane = 1 + tok
            splane = jnp.where(plane >= RING, plane - RING, plane)
            out = []
            for i in range(2):
                v = noise_v[i, k >> 3, pl.ds((k & 7) * NS, NS)]
                m = _monotone_key(plsc.bitcast(v, jnp.int32))
                keep = jnp.logical_or(
                    m < tv[i], jnp.logical_and(m == tv[i], tok <= jv[i]))
                keep_i = jnp.where(keep, 1, 0).astype(jnp.int32)
                cum = plsc.cumsum(keep_i)
                plsc.store_scatter(colblk, [carry[i] + cum, bv[i]],
                                   splane, mask=keep)
                out.append(carry[i] + cum[15])
            return tuple(out)

        lax.fori_loop(0, L // NS, chunk, (jnp.int32(0), jnp.int32(0)))

    pltpu.sync_copy(colblk, ksp.at[:, pl.ds(sid * B_PER_S, B_PER_S)])
    plsc.subcore_barrier()

    zeros16 = jnp.zeros((NS,), jnp.int32)
    lane_g = [g * NS + lanes for g in range(B // NS)]
    for ff in range(F_PER_W):
        f = wid * F_PER_W + ff
        hin = hin0 if ff == 0 else {}

        def fire_chunk(c, hin=hin, f=f):
            fire_chunk_for(f, hin, c)

        hk = [None] * NBLK

        def fire_kwin(J):
            hk[J] = pltpu.async_copy(ksp.at[pl.ds(J * OBLK, OBLK)],
                                     kwin.at[J % 2], sem_k)

        if ff != 0:
            for c in range(9):
                fire_chunk(c)
        fire_kwin(0)
        for c in range(9):
            hin[c].wait()
        hout = [None] * NBLK
        for J in range(NBLK):
            nrows = OBLK if J < NBLK - 1 else P_OUT - (NBLK - 1) * OBLK
            hk[J].wait()
            if J + 1 < NBLK:
                fire_kwin(J + 1)
            if J >= 1:
                hin[J + 8].wait()
            if J + 9 < NCHUNK:
                fire_chunk(J + 9)
            if J >= 2:
                hout[J - 2].wait()
            kb = J % 2

            def one_row(jj):
                for g in range(B // NS):
                    slot = kwin[kb, jj, pl.ds(g * NS, NS)]
                    vals = plsc.load_gather(ring, [slot, zeros16, lane_g[g]])
                    obuf[kb, jj, 0, pl.ds(g * NS, NS)] = vals

            if nrows == OBLK:
                def row2(jj, _):
                    one_row(2 * jj)
                    one_row(2 * jj + 1)
                    return 0

                lax.fori_loop(0, OBLK // 2, row2, 0)
            else:
                def row1(jj, _):
                    one_row(jj)
                    return 0

                lax.fori_loop(0, nrows, row1, 0)
            hout[J] = pltpu.async_copy(
                obuf.at[kb].at[pl.ds(0, nrows)],
                out_hbm.at[pl.ds(J * OBLK, nrows), pl.ds(f, 1), :], sem_out)
        hout[NBLK - 2].wait()
        hout[NBLK - 1].wait()


def _sc_gather(noise, thr, need, x_t):
    mesh = plsc.VectorSubcoreMesh(core_axis_name="c", subcore_axis_name="s")
    k = functools.partial(
        pl.kernel,
        mesh=mesh,
        out_type=jax.ShapeDtypeStruct((P_OUT, D, B), jnp.float32),
        scratch_types=[
            pltpu.VMEM((2, L // 128, 128), jnp.float32),
            pltpu.VMEM((B_PER_S, NS), jnp.int32),
            pltpu.VMEM((B_PER_S, NS), jnp.int32),
            pltpu.VMEM((KROWS, B_PER_S), jnp.int32),
            pltpu.VMEM((RING, 1, B), jnp.float32),
            pltpu.VMEM((2, OBLK, B), jnp.int32),
            pltpu.VMEM((2, OBLK, 1, B), jnp.float32),
            pltpu.VMEM_SHARED((KROWS, B), jnp.int32),
            pltpu.SemaphoreType.DMA,
            pltpu.SemaphoreType.DMA,
            pltpu.SemaphoreType.DMA,
        ],
        compiler_params=pltpu.CompilerParams(
            needs_layout_passes=False, use_tc_tiling_on_sc=False),
    )(_sc_body)
    return k(noise, thr, need, x_t)


def kernel(x):
    noise = jax.random.normal(jax.random.key(1), (B, L), dtype=jnp.float32)
    thr, need = _tc_select(noise)
    x_t = jnp.transpose(x, (1, 2, 0))
    out_t = _sc_gather(noise.reshape(B, L // 128, 128), thr, need, x_t)
    return jnp.transpose(out_t, (2, 0, 1))
